# baseline (speedup 1.0000x reference)
import functools

import jax
import jax.numpy as jnp
from jax import lax
from jax.experimental import pallas as pl
from jax.experimental.pallas import tpu as pltpu
from jax.experimental.pallas import tpu_sc as plsc

E = 8
K = 2
D = 1024
DFF = 512
T = 2048
A = T * K
BM = 256
NBLK = (A + E * (BM - 1) + BM - 1) // BM
P = NBLK * BM
TB = 512

NC = 2
NS = 16
NW = NC * NS
APW = A // NW
TPW = T // NW

_f32 = jnp.float32
_i32 = jnp.int32


def _router_body(x_ref, gw_ref, logits_ref, e_ref, r_ref, w_ref, off_ref,
                 bexp_ref, x16_ref, carry_ref):
    pid = pl.program_id(0)

    @pl.when(pid == 0)
    def _():
        carry_ref[...] = jnp.zeros_like(carry_ref)

    xb = x_ref[...]
    u = lax.bitcast_convert_type(xb, _i32)
    rnd = (u + 0x7FFF + (lax.shift_right_logical(u, 16) & 1))
    lo = lax.shift_right_logical(rnd[:, :D // 2], 16)
    hi = rnd[:, D // 2:] & jnp.int32(-65536)
    x16_ref[...] = lo | hi
    gw = gw_ref[...]
    logits = lax.dot_general(xb, gw, (((1,), (1,)), ((), ())),
                             preferred_element_type=_f32)
    logits_ref[...] = logits

    m = jnp.max(logits, axis=-1, keepdims=True)
    ex = jnp.exp(logits - m)
    probs = ex / jnp.sum(ex, axis=-1, keepdims=True)

    iota_e = lax.broadcasted_iota(_i32, probs.shape, 1)
    p0 = jnp.max(probs, axis=-1, keepdims=True)
    e0 = jnp.min(jnp.where(probs >= p0, iota_e, E), axis=-1, keepdims=True)
    oh0 = iota_e == e0
    probs2 = jnp.where(oh0, -1.0, probs)
    p1 = jnp.max(probs2, axis=-1, keepdims=True)
    e1 = jnp.min(jnp.where(probs2 >= p1, iota_e, E), axis=-1, keepdims=True)
    oh1 = iota_e == e1

    denom = p0 + p1 + 1e-20
    w0 = p0 / denom / K
    w1 = p1 / denom / K

    oh0f = oh0.astype(_f32)
    oh1f = oh1.astype(_f32)
    ohsum = oh0f + oh1f
    ir = lax.broadcasted_iota(_i32, (TB, TB), 0)
    ic = lax.broadcasted_iota(_i32, (TB, TB), 1)
    ltri = (ir > ic).astype(_f32)
    csum = lax.dot_general(ltri, ohsum, (((1,), (0,)), ((), ())),
                           preferred_element_type=_f32)
    cfull = csum + carry_ref[...]
    r0 = jnp.sum(cfull * oh0f, axis=-1, keepdims=True)
    r1 = jnp.sum(cfull * oh1f, axis=-1, keepdims=True)
    carry_new = carry_ref[...] + jnp.sum(ohsum, axis=0, keepdims=True)
    carry_ref[...] = carry_new

    e_ref[...] = jnp.concatenate([e0, e1], axis=1)
    r_ref[...] = jnp.concatenate([r0, r1], axis=1).astype(_i32)
    w_ref[...] = jnp.concatenate([w0, w1], axis=1)

    padded_row = jnp.floor((carry_new + (BM - 1)) / BM) * BM
    p8 = jnp.broadcast_to(padded_row, (E, E))
    ri8 = lax.broadcasted_iota(_i32, (E, E), 0)
    ci8 = lax.broadcasted_iota(_i32, (E, E), 1)
    pcol = jnp.sum(p8 * (ri8 == ci8).astype(_f32), axis=1,
                   keepdims=True)
    p8t = jnp.broadcast_to(pcol, (E, E))
    ends_row = jnp.sum(p8t * (ri8 <= ci8).astype(_f32), axis=0,
                       keepdims=True)
    off_row = ends_row - padded_row
    off16 = jnp.concatenate([off_row, jnp.zeros((1, 16 - E), _f32)],
                            axis=1)
    off_ref[...] = jnp.broadcast_to(off16, (8, 16)).astype(_i32)
    ends24 = jnp.broadcast_to(ends_row, (NBLK, E))
    bstart = lax.broadcasted_iota(_i32, (NBLK, E), 0).astype(_f32) * BM
    nends = jnp.sum((ends24 <= bstart).astype(_f32), axis=1,
                    keepdims=True)
    bexp = jnp.minimum(nends, E - 1)
    bexp_ref[...] = jnp.broadcast_to(bexp, (NBLK, E)).astype(_i32)


def _router_call(x2d, gate_w, interpret=False):
    return pl.pallas_call(
        _router_body,
        grid=(T // TB,),
        in_specs=[
            pl.BlockSpec((TB, D), lambda i: (i, 0)),
            pl.BlockSpec((E, D), lambda i: (0, 0)),
        ],
        out_specs=[
            pl.BlockSpec((TB, E), lambda i: (i, 0)),
            pl.BlockSpec((TB, K), lambda i: (i, 0)),
            pl.BlockSpec((TB, K), lambda i: (i, 0)),
            pl.BlockSpec((TB, K), lambda i: (i, 0)),
            pl.BlockSpec((8, 16), lambda i: (0, 0)),
            pl.BlockSpec((NBLK, E), lambda i: (0, 0)),
            pl.BlockSpec((TB, D // 2), lambda i: (i, 0)),
        ],
        out_shape=[
            jax.ShapeDtypeStruct((T, E), _f32),
            jax.ShapeDtypeStruct((T, K), _i32),
            jax.ShapeDtypeStruct((T, K), _i32),
            jax.ShapeDtypeStruct((T, K), _f32),
            jax.ShapeDtypeStruct((8, 16), _i32),
            jax.ShapeDtypeStruct((NBLK, E), _i32),
            jax.ShapeDtypeStruct((T, D // 2), _i32),
        ],
        scratch_shapes=[pltpu.VMEM((1, E), _f32)],
        interpret=interpret,
    )(x2d, gate_w)


@functools.cache
def _make_dispatch():
    mesh = plsc.VectorSubcoreMesh(core_axis_name="c", subcore_axis_name="s",
                                  num_cores=NC, num_subcores=NS)

    @functools.partial(
        pl.kernel,
        out_type=(jax.ShapeDtypeStruct((P, D // 2), _i32),
                  jax.ShapeDtypeStruct((A,), _i32)),
        mesh=mesh,
        scratch_types=[
            pltpu.VMEM((APW,), _i32),
            pltpu.VMEM((APW,), _i32),
            pltpu.VMEM((16,), _i32),
            pltpu.VMEM((APW,), _i32),
            pltpu.VMEM((2, 32), _i32),
            pltpu.VMEM((2, 32), _i32),
            pltpu.VMEM((32, D // 2), _i32),
            pltpu.VMEM((32, D // 2), _i32),
            pltpu.SemaphoreType.DMA,
            pltpu.SemaphoreType.DMA,
        ],
        compiler_params=pltpu.CompilerParams(needs_layout_passes=False),
    )
    def _dispatch(eflat_hbm, rflat_hbm, off_hbm, x_hbm, xbuf_hbm, pos_hbm,
                  e_v, r_v, off_v, posf_v, pe2d_v, po2d_v, rows0_v, rows1_v,
                  gsem, ssem):
        wid = lax.axis_index("s") * NC + lax.axis_index("c")
        base = wid * APW
        tbase = wid * TPW
        pltpu.sync_copy(eflat_hbm.at[pl.ds(base, APW)], e_v)
        pltpu.sync_copy(rflat_hbm.at[pl.ds(base, APW)], r_v)
        pltpu.sync_copy(off_hbm.at[0], off_v)
        iota16 = lax.iota(_i32, 16)
        for j in range(APW // 16):
            ev = e_v[pl.ds(j * 16, 16)]
            rv = r_v[pl.ds(j * 16, 16)]
            posf_v[pl.ds(j * 16, 16)] = plsc.load_gather(off_v, [ev]) + rv
        for c in range(2):
            for h in range(2):
                gi = c * 64 + 2 * (h * 16 + iota16)
                pe2d_v[c, pl.ds(h * 16, 16)] = plsc.load_gather(posf_v, [gi])
                po2d_v[c, pl.ds(h * 16, 16)] = plsc.load_gather(posf_v,
                                                                [gi + 1])
        pltpu.sync_copy(posf_v, pos_hbm.at[pl.ds(base, APW)])
        bufs = (rows0_v, rows1_v)
        g0 = pltpu.async_copy(x_hbm.at[pl.ds(tbase, 32)], rows0_v, gsem)
        g1 = pltpu.async_copy(x_hbm.at[pl.ds(tbase + 32, 32)], rows1_v, gsem)
        gets = (g0, g1)
        puts = []
        for c in range(2):
            gets[c].wait()
            puts.append(pltpu.async_copy(bufs[c], xbuf_hbm.at[pe2d_v.at[c]],
                                         ssem))
            puts.append(pltpu.async_copy(bufs[c], xbuf_hbm.at[po2d_v.at[c]],
                                         ssem))
        for p in puts:
            p.wait()

    return _dispatch


def _gmm_body(be_ref, x_ref, w1_ref, b1_ref, w2_ref, b2_ref, y_ref):
    b = pl.program_id(0)
    e = be_ref[b, 0]
    xp = x_ref[...]
    xlo = lax.bitcast_convert_type(lax.shift_left(xp, 16), _f32)
    xhi = lax.bitcast_convert_type(xp & jnp.int32(-65536), _f32)
    xb = jnp.concatenate([xlo, xhi], axis=1).astype(jnp.bfloat16)
    w1 = w1_ref[pl.ds(e, 1)][0]
    w2 = w2_ref[pl.ds(e, 1)][0]
    b1 = b1_ref[pl.ds(e, 1)]
    b2 = b2_ref[pl.ds(e, 1)]
    h = lax.dot_general(xb, w1, (((1,), (0,)), ((), ())),
                        preferred_element_type=_f32) + b1
    a = (h * jax.nn.sigmoid(h)).astype(jnp.bfloat16)
    y = lax.dot_general(a, w2, (((1,), (0,)), ((), ())),
                        preferred_element_type=_f32) + b2
    u = lax.bitcast_convert_type(y, _i32)
    rnd = (u + 0x7FFF + (lax.shift_right_logical(u, 16) & 1))
    y_ref[...] = (lax.shift_right_logical(rnd[:, :D // 2], 16)
                  | (rnd[:, D // 2:] & jnp.int32(-65536)))


def _gmm_call(block_expert, xbuf, W1, b1, W2, b2, interpret=False):
    grid_spec = pltpu.PrefetchScalarGridSpec(
        num_scalar_prefetch=1,
        grid=(NBLK,),
        in_specs=[
            pl.BlockSpec((BM, D // 2), lambda b, be: (b, 0)),
            pl.BlockSpec((E, D, DFF), lambda b, be: (0, 0, 0)),
            pl.BlockSpec((E, DFF), lambda b, be: (0, 0)),
            pl.BlockSpec((E, DFF, D), lambda b, be: (0, 0, 0)),
            pl.BlockSpec((E, D), lambda b, be: (0, 0)),
        ],
        out_specs=pl.BlockSpec((BM, D // 2), lambda b, be: (b, 0)),
    )
    return pl.pallas_call(
        _gmm_body,
        grid_spec=grid_spec,
        out_shape=jax.ShapeDtypeStruct((P, D // 2), _i32),
        compiler_params=pltpu.CompilerParams(
            vmem_limit_bytes=100 * 1024 * 1024),
        interpret=interpret,
    )(block_expert, xbuf, W1.astype(jnp.bfloat16), b1,
      W2.astype(jnp.bfloat16), b2)


@functools.cache
def _make_combine():
    mesh = plsc.VectorSubcoreMesh(core_axis_name="c", subcore_axis_name="s",
                                  num_cores=NC, num_subcores=NS)

    @functools.partial(
        pl.kernel,
        out_type=jax.ShapeDtypeStruct((T, D), _f32),
        mesh=mesh,
        scratch_types=[
            pltpu.VMEM((2 * TPW,), _i32),
            pltpu.VMEM((2 * TPW,), _f32),
            pltpu.VMEM((4, 16), _i32),
            pltpu.VMEM((4, 16), _i32),
            pltpu.VMEM((4, 16), _f32),
            pltpu.VMEM((4, 16), _f32),
            pltpu.VMEM((2, 16, D // 2), _i32),
            pltpu.VMEM((2, 16, D // 2), _i32),
            pltpu.VMEM((2, 16, D), _f32),
            pltpu.SemaphoreType.DMA,
            pltpu.SemaphoreType.DMA,
        ],
        compiler_params=pltpu.CompilerParams(needs_layout_passes=False),
    )
    def _combine(ybuf_hbm, pos_hbm, w_hbm, out_hbm,
                 pos_v, w_v, p0_v, p1_v, w0_v, w1_v, rows0_v, rows1_v, out_v,
                 gsem, osem):
        wid = lax.axis_index("s") * NC + lax.axis_index("c")
        abase = wid * (2 * TPW)
        tbase = wid * TPW
        pltpu.sync_copy(pos_hbm.at[pl.ds(abase, 2 * TPW)], pos_v)
        pltpu.sync_copy(w_hbm.at[pl.ds(abase, 2 * TPW)], w_v)
        iota16 = lax.iota(_i32, 16)
        nch = TPW // 16
        for c in range(nch):
            gi = c * 32 + 2 * iota16
            p0_v[c, :] = plsc.load_gather(pos_v, [gi])
            p1_v[c, :] = plsc.load_gather(pos_v, [gi + 1])
            w0_v[c, :] = plsc.load_gather(w_v, [gi])
            w1_v[c, :] = plsc.load_gather(w_v, [gi + 1])

        def fetch(c):
            b = c % 2
            return (pltpu.async_copy(ybuf_hbm.at[p0_v.at[c]], rows0_v.at[b],
                                     gsem),
                    pltpu.async_copy(ybuf_hbm.at[p1_v.at[c]], rows1_v.at[b],
                                     gsem))

        pend = fetch(0)
        owaits = [None, None]
        for c in range(nch):
            b = c % 2
            pend[0].wait()
            pend[1].wait()
            if c + 1 < nch:
                pend = fetch(c + 1)
            if owaits[b] is not None:
                owaits[b].wait()

            def tok_body(t, carry, _b=b):
                ts = iota16 * 0 + t
                w0s = plsc.load_gather(w0_v.at[c], [ts])
                w1s = plsc.load_gather(w1_v.at[c], [ts])
                w0b = plsc.pack(w0s, w0s,
                                format=plsc.PackFormat.INTERLEAVED)
                w1b = plsc.pack(w1s, w1s,
                                format=plsc.PackFormat.INTERLEAVED)
                for j in range(D // 2 // 16):
                    sl = pl.ds(j * 16, 16)
                    slh = pl.ds(D // 2 + j * 16, 16)
                    r0 = plsc.bitcast(rows0_v[_b, t, sl], jnp.bfloat16)
                    r1 = plsc.bitcast(rows1_v[_b, t, sl], jnp.bfloat16)
                    acc = w0b * r0 + w1b * r1
                    lo, hi = plsc.unpack(
                        acc, format=plsc.PackFormat.INTERLEAVED)
                    out_v[_b, t, sl] = lo
                    out_v[_b, t, slh] = hi
                return carry

            lax.fori_loop(0, 16, tok_body, 0)
            owaits[b] = pltpu.async_copy(
                out_v.at[b], out_hbm.at[pl.ds(tbase + c * 16, 16)], osem)
        for ow in owaits:
            if ow is not None:
                ow.wait()

    return _combine


def kernel(x, gate_w, W1, b1, W2, b2):
    orig_shape = x.shape
    x2d = x.reshape(-1, x.shape[-1]).astype(_f32)

    logits, eidx, ranks, wts, off16, bexp, x16 = _router_call(x2d, gate_w)

    eflat = eidx.reshape(A)
    rflat = ranks.reshape(A)
    wflat = wts.reshape(A)

    xbuf, pos = _make_dispatch()(eflat, rflat, off16, x16)
    ybuf = _gmm_call(bexp, xbuf, W1, b1, W2, b2)
    out2d = _make_combine()(ybuf, pos, wflat)

    return out2d.reshape(orig_shape), logits


# speedup vs baseline: 1.1531x; 1.1531x over previous
import functools

import jax
import jax.numpy as jnp
from jax import lax
from jax.experimental import pallas as pl
from jax.experimental.pallas import tpu as pltpu
from jax.experimental.pallas import tpu_sc as plsc

E = 8
K = 2
D = 1024
DFF = 512
T = 2048
A = T * K
BM = 256
NBLK = (A + E * (BM - 1) + BM - 1) // BM
P = NBLK * BM
TB = 512

NC = 2
NS = 16
NW = NC * NS
APW = A // NW
TPW = T // NW

_f32 = jnp.float32
_i32 = jnp.int32


def _router_body(x_ref, gw_ref, logits_ref, erw_ref, off_ref,
                 bexp_ref, x16_ref, carry_ref):
    pid = pl.program_id(0)

    @pl.when(pid == 0)
    def _():
        carry_ref[...] = jnp.zeros_like(carry_ref)

    xb = x_ref[...]
    u = lax.bitcast_convert_type(xb, _i32)
    rnd = (u + 0x7FFF + (lax.shift_right_logical(u, 16) & 1))
    lo = lax.shift_right_logical(rnd[:, :D // 2], 16)
    hi = rnd[:, D // 2:] & jnp.int32(-65536)
    x16_ref[...] = lo | hi
    gw = gw_ref[...]
    logits = lax.dot_general(xb, gw, (((1,), (1,)), ((), ())),
                             preferred_element_type=_f32)
    logits_ref[...] = logits

    m = jnp.max(logits, axis=-1, keepdims=True)
    ex = jnp.exp(logits - m)
    probs = ex / jnp.sum(ex, axis=-1, keepdims=True)

    iota_e = lax.broadcasted_iota(_i32, probs.shape, 1)
    p0 = jnp.max(probs, axis=-1, keepdims=True)
    e0 = jnp.min(jnp.where(probs >= p0, iota_e, E), axis=-1, keepdims=True)
    oh0 = iota_e == e0
    probs2 = jnp.where(oh0, -1.0, probs)
    p1 = jnp.max(probs2, axis=-1, keepdims=True)
    e1 = jnp.min(jnp.where(probs2 >= p1, iota_e, E), axis=-1, keepdims=True)
    oh1 = iota_e == e1

    denom = p0 + p1 + 1e-20
    w0 = p0 / denom / K
    w1 = p1 / denom / K

    oh0f = oh0.astype(_f32)
    oh1f = oh1.astype(_f32)
    ohsum = oh0f + oh1f
    ir = lax.broadcasted_iota(_i32, (TB, TB), 0)
    ic = lax.broadcasted_iota(_i32, (TB, TB), 1)
    ltri = (ir > ic).astype(_f32)
    csum = lax.dot_general(ltri, ohsum, (((1,), (0,)), ((), ())),
                           preferred_element_type=_f32)
    cfull = csum + carry_ref[...]
    r0 = jnp.sum(cfull * oh0f, axis=-1, keepdims=True)
    r1 = jnp.sum(cfull * oh1f, axis=-1, keepdims=True)
    carry_new = carry_ref[...] + jnp.sum(ohsum, axis=0, keepdims=True)
    carry_ref[...] = carry_new

    def _packed(ev, rv, wv):
        uw = lax.bitcast_convert_type(wv, _i32)
        rw = ((uw + 0x7FFF + (lax.shift_right_logical(uw, 16) & 1))
              & jnp.int32(-65536))
        return ev | lax.shift_left(rv.astype(_i32), 3) | rw

    erw_ref[...] = jnp.concatenate(
        [_packed(e0, r0, w0), _packed(e1, r1, w1)], axis=1)

    padded_row = jnp.floor((carry_new + (BM - 1)) / BM) * BM
    p8 = jnp.broadcast_to(padded_row, (E, E))
    ri8 = lax.broadcasted_iota(_i32, (E, E), 0)
    ci8 = lax.broadcasted_iota(_i32, (E, E), 1)
    pcol = jnp.sum(p8 * (ri8 == ci8).astype(_f32), axis=1,
                   keepdims=True)
    p8t = jnp.broadcast_to(pcol, (E, E))
    ends_row = jnp.sum(p8t * (ri8 <= ci8).astype(_f32), axis=0,
                       keepdims=True)
    off_row = ends_row - padded_row
    off16 = jnp.concatenate([off_row, jnp.zeros((1, 16 - E), _f32)],
                            axis=1)
    off_ref[...] = jnp.broadcast_to(off16, (8, 16)).astype(_i32)
    ends24 = jnp.broadcast_to(ends_row, (NBLK, E))
    bstart = lax.broadcasted_iota(_i32, (NBLK, E), 0).astype(_f32) * BM
    nends = jnp.sum((ends24 <= bstart).astype(_f32), axis=1,
                    keepdims=True)
    bexp = jnp.minimum(nends, E - 1)
    bexp_ref[...] = jnp.broadcast_to(bexp, (NBLK, E)).astype(_i32)


def _router_call(x2d, gate_w, interpret=False):
    return pl.pallas_call(
        _router_body,
        grid=(T // TB,),
        in_specs=[
            pl.BlockSpec((TB, D), lambda i: (i, 0)),
            pl.BlockSpec((E, D), lambda i: (0, 0)),
        ],
        out_specs=[
            pl.BlockSpec((TB, E), lambda i: (i, 0)),
            pl.BlockSpec((TB, K), lambda i: (i, 0)),
            pl.BlockSpec((8, 16), lambda i: (0, 0)),
            pl.BlockSpec((NBLK, E), lambda i: (0, 0)),
            pl.BlockSpec((TB, D // 2), lambda i: (i, 0)),
        ],
        out_shape=[
            jax.ShapeDtypeStruct((T, E), _f32),
            jax.ShapeDtypeStruct((T, K), _i32),
            jax.ShapeDtypeStruct((8, 16), _i32),
            jax.ShapeDtypeStruct((NBLK, E), _i32),
            jax.ShapeDtypeStruct((T, D // 2), _i32),
        ],
        scratch_shapes=[pltpu.VMEM((1, E), _f32)],
        interpret=interpret,
    )(x2d, gate_w)


@functools.cache
def _make_dispatch():
    mesh = plsc.VectorSubcoreMesh(core_axis_name="c", subcore_axis_name="s",
                                  num_cores=NC, num_subcores=NS)

    @functools.partial(
        pl.kernel,
        out_type=(jax.ShapeDtypeStruct((P, D // 2), _i32),
                  jax.ShapeDtypeStruct((A,), _i32)),
        mesh=mesh,
        scratch_types=[
            pltpu.VMEM((APW,), _i32),
            pltpu.VMEM((16,), _i32),
            pltpu.VMEM((APW,), _i32),
            pltpu.VMEM((APW,), _i32),
            pltpu.VMEM((2, 32), _i32),
            pltpu.VMEM((2, 32), _i32),
            pltpu.VMEM((32, D // 2), _i32),
            pltpu.VMEM((32, D // 2), _i32),
            pltpu.SemaphoreType.DMA,
            pltpu.SemaphoreType.DMA,
        ],
        compiler_params=pltpu.CompilerParams(needs_layout_passes=False),
    )
    def _dispatch(erw_hbm, off_hbm, x_hbm, xbuf_hbm, pos_hbm,
                  erw_v, off_v, posc_v, posf_v, pe2d_v, po2d_v,
                  rows0_v, rows1_v, gsem, ssem):
        wid = lax.axis_index("s") * NC + lax.axis_index("c")
        base = wid * APW
        tbase = wid * TPW
        pltpu.sync_copy(erw_hbm.at[pl.ds(base, APW)], erw_v)
        pltpu.sync_copy(off_hbm.at[0], off_v)
        iota16 = lax.iota(_i32, 16)
        m7 = jnp.full((16,), 7, _i32)
        m13 = jnp.full((16,), 0x1FFF, _i32)
        mw = jnp.full((16,), -65536, _i32)
        sh3 = jnp.full((16,), 3, _i32)
        for j in range(APW // 16):
            v = erw_v[pl.ds(j * 16, 16)]
            ev = v & m7
            rv = lax.shift_right_logical(v, sh3) & m13
            pos = plsc.load_gather(off_v, [ev]) + rv
            posc_v[pl.ds(j * 16, 16)] = pos
            posf_v[pl.ds(j * 16, 16)] = pos | (v & mw)
        for c in range(2):
            for h in range(2):
                gi = c * 64 + 2 * (h * 16 + iota16)
                pe2d_v[c, pl.ds(h * 16, 16)] = plsc.load_gather(posc_v, [gi])
                po2d_v[c, pl.ds(h * 16, 16)] = plsc.load_gather(posc_v,
                                                                [gi + 1])
        pltpu.sync_copy(posf_v, pos_hbm.at[pl.ds(base, APW)])
        bufs = (rows0_v, rows1_v)
        g0 = pltpu.async_copy(x_hbm.at[pl.ds(tbase, 32)], rows0_v, gsem)
        g1 = pltpu.async_copy(x_hbm.at[pl.ds(tbase + 32, 32)], rows1_v, gsem)
        gets = (g0, g1)
        puts = []
        for c in range(2):
            gets[c].wait()
            puts.append(pltpu.async_copy(bufs[c], xbuf_hbm.at[pe2d_v.at[c]],
                                         ssem))
            puts.append(pltpu.async_copy(bufs[c], xbuf_hbm.at[po2d_v.at[c]],
                                         ssem))
        for p in puts:
            p.wait()

    return _dispatch


def _gmm_body(be_ref, x_ref, w1_ref, b1_ref, w2_ref, b2_ref, y_ref):
    b = pl.program_id(0)
    e = be_ref[b, 0]
    xp = x_ref[...]
    xlo = lax.bitcast_convert_type(lax.shift_left(xp, 16), _f32)
    xhi = lax.bitcast_convert_type(xp & jnp.int32(-65536), _f32)
    xb = jnp.concatenate([xlo, xhi], axis=1)
    w1 = w1_ref[pl.ds(e, 1)][0]
    w2 = w2_ref[pl.ds(e, 1)][0]
    b1 = b1_ref[pl.ds(e, 1)]
    b2 = b2_ref[pl.ds(e, 1)]
    h = lax.dot_general(xb, w1, (((1,), (0,)), ((), ())),
                        preferred_element_type=_f32) + b1
    a = h * jax.nn.sigmoid(h)
    y = lax.dot_general(a, w2, (((1,), (0,)), ((), ())),
                        preferred_element_type=_f32) + b2
    u = lax.bitcast_convert_type(y, _i32)
    rnd = (u + 0x7FFF + (lax.shift_right_logical(u, 16) & 1))
    y_ref[...] = (lax.shift_right_logical(rnd[:, :D // 2], 16)
                  | (rnd[:, D // 2:] & jnp.int32(-65536)))


def _gmm_call(block_expert, xbuf, W1, b1, W2, b2, interpret=False):
    grid_spec = pltpu.PrefetchScalarGridSpec(
        num_scalar_prefetch=1,
        grid=(NBLK,),
        in_specs=[
            pl.BlockSpec((BM, D // 2), lambda b, be: (b, 0)),
            pl.BlockSpec((E, D, DFF), lambda b, be: (0, 0, 0)),
            pl.BlockSpec((E, DFF), lambda b, be: (0, 0)),
            pl.BlockSpec((E, DFF, D), lambda b, be: (0, 0, 0)),
            pl.BlockSpec((E, D), lambda b, be: (0, 0)),
        ],
        out_specs=pl.BlockSpec((BM, D // 2), lambda b, be: (b, 0)),
    )
    return pl.pallas_call(
        _gmm_body,
        grid_spec=grid_spec,
        out_shape=jax.ShapeDtypeStruct((P, D // 2), _i32),
        compiler_params=pltpu.CompilerParams(
            vmem_limit_bytes=100 * 1024 * 1024),
        interpret=interpret,
    )(block_expert, xbuf, W1, b1, W2, b2)


@functools.cache
def _make_combine():
    mesh = plsc.VectorSubcoreMesh(core_axis_name="c", subcore_axis_name="s",
                                  num_cores=NC, num_subcores=NS)

    @functools.partial(
        pl.kernel,
        out_type=jax.ShapeDtypeStruct((T, D), _f32),
        mesh=mesh,
        scratch_types=[
            pltpu.VMEM((2 * TPW,), _i32),
            pltpu.VMEM((4, 16), _i32),
            pltpu.VMEM((4, 16), _i32),
            pltpu.VMEM((4, 16), _i32),
            pltpu.VMEM((4, 16), _i32),
            pltpu.VMEM((2, 16, D // 2), _i32),
            pltpu.VMEM((2, 16, D // 2), _i32),
            pltpu.VMEM((2, 16, D), _f32),
            pltpu.SemaphoreType.DMA,
            pltpu.SemaphoreType.DMA,
        ],
        compiler_params=pltpu.CompilerParams(needs_layout_passes=False),
    )
    def _combine(ybuf_hbm, pos_hbm, out_hbm,
                 pos_v, p0_v, p1_v, w0_v, w1_v, rows0_v, rows1_v, out_v,
                 gsem, osem):
        wid = lax.axis_index("s") * NC + lax.axis_index("c")
        abase = wid * (2 * TPW)
        tbase = wid * TPW
        pltpu.sync_copy(pos_hbm.at[pl.ds(abase, 2 * TPW)], pos_v)
        iota16 = lax.iota(_i32, 16)
        mlow = jnp.full((16,), 0xFFFF, _i32)
        mw = jnp.full((16,), -65536, _i32)
        sh16 = jnp.full((16,), 16, _i32)
        nch = TPW // 16
        for c in range(nch):
            gi = c * 32 + 2 * iota16
            pw0 = plsc.load_gather(pos_v, [gi])
            pw1 = plsc.load_gather(pos_v, [gi + 1])
            p0_v[c, :] = pw0 & mlow
            p1_v[c, :] = pw1 & mlow
            wb0 = pw0 & mw
            wb1 = pw1 & mw
            w0_v[c, :] = wb0 | lax.shift_right_logical(wb0, sh16)
            w1_v[c, :] = wb1 | lax.shift_right_logical(wb1, sh16)

        def fetch(c):
            b = c % 2
            return (pltpu.async_copy(ybuf_hbm.at[p0_v.at[c]], rows0_v.at[b],
                                     gsem),
                    pltpu.async_copy(ybuf_hbm.at[p1_v.at[c]], rows1_v.at[b],
                                     gsem))

        pend = fetch(0)
        owaits = [None, None]
        for c in range(nch):
            b = c % 2
            pend[0].wait()
            pend[1].wait()
            if c + 1 < nch:
                pend = fetch(c + 1)
            if owaits[b] is not None:
                owaits[b].wait()

            def tok_body(t, carry, _b=b):
                ts = iota16 * 0 + t
                w0b = plsc.bitcast(plsc.load_gather(w0_v.at[c], [ts]),
                                   jnp.bfloat16)
                w1b = plsc.bitcast(plsc.load_gather(w1_v.at[c], [ts]),
                                   jnp.bfloat16)
                for j in range(D // 2 // 16):
                    sl = pl.ds(j * 16, 16)
                    slh = pl.ds(D // 2 + j * 16, 16)
                    r0 = plsc.bitcast(rows0_v[_b, t, sl], jnp.bfloat16)
                    r1 = plsc.bitcast(rows1_v[_b, t, sl], jnp.bfloat16)
                    acc = w0b * r0 + w1b * r1
                    lo, hi = plsc.unpack(
                        acc, format=plsc.PackFormat.INTERLEAVED)
                    out_v[_b, t, sl] = lo
                    out_v[_b, t, slh] = hi
                return carry

            lax.fori_loop(0, 16, tok_body, 0)
            owaits[b] = pltpu.async_copy(
                out_v.at[b], out_hbm.at[pl.ds(tbase + c * 16, 16)], osem)
        for ow in owaits:
            if ow is not None:
                ow.wait()

    return _combine


def kernel(x, gate_w, W1, b1, W2, b2):
    orig_shape = x.shape
    x2d = x.reshape(-1, x.shape[-1]).astype(_f32)

    logits, erw, off16, bexp, x16 = _router_call(x2d, gate_w)

    xbuf, pos = _make_dispatch()(erw.reshape(A), off16, x16)
    ybuf = _gmm_call(bexp, xbuf, W1, b1, W2, b2)
    out2d = _make_combine()(ybuf, pos)

    return out2d.reshape(orig_shape), logits
